# baseline (speedup 1.0000x reference)
import functools

import jax
import jax.numpy as jnp
from jax import lax
from jax.experimental import pallas as pl
from jax.experimental.pallas import tpu as pltpu
from jax.experimental.pallas import tpu_sc as plsc

B, S, D, H, L = 1, 2048, 1024, 16, 2
DH = D // H
M, K = 4096, 32
FF = 4 * D
SCALE = DH ** -0.5
QB = 256
NQ = S // QB
HP = 2

NR = H * S
MW = M + 16
NWORK = 32
RPW = NR // NWORK
RB = 8
NB = RPW // RB
CAP = 1088
OW = K + 16
KEY_INF = 0x7F800000
MASK31 = 0x7FFFFFFF


def _qkv_kernel(x_ref, g_ref, wq_ref, wk_ref, wv_ref, q_ref, k_ref, v_ref):
    x = x_ref[...]
    mu = jnp.mean(x, axis=-1, keepdims=True)
    var = jnp.mean((x - mu) ** 2, axis=-1, keepdims=True)
    h = (x - mu) * jax.lax.rsqrt(var + 1e-5) * g_ref[...]
    q_ref[...] = jnp.dot(h, wq_ref[...], preferred_element_type=jnp.float32)
    k_ref[...] = jnp.dot(h, wk_ref[...], preferred_element_type=jnp.float32)
    v_ref[...] = jnp.dot(h, wv_ref[...], preferred_element_type=jnp.float32)


def _qkv(x, g, wq, wk, wv):
    shp = jax.ShapeDtypeStruct((S, D), jnp.float32)
    full = pl.BlockSpec((D, D), lambda i: (0, 0))
    row = pl.BlockSpec((QB, D), lambda i: (i, 0))
    return pl.pallas_call(
        _qkv_kernel,
        grid=(NQ,),
        in_specs=[row, pl.BlockSpec((1, D), lambda i: (0, 0)), full, full, full],
        out_specs=[row, row, row],
        out_shape=[shp, shp, shp],
    )(x, g, wq, wk, wv)


def _causal_scores(q, k_all, i):
    s = jax.lax.dot_general(q, k_all, (((1,), (1,)), ((), ())),
                            preferred_element_type=jnp.float32) * SCALE
    rows = i * QB + jax.lax.broadcasted_iota(jnp.int32, (QB, S), 0)
    cols = jax.lax.broadcasted_iota(jnp.int32, (QB, S), 1)
    return jnp.where(cols <= rows, s, -1e9)


def _attn_kernel(q_ref, k_ref, v_ref, o_ref):
    i = pl.program_id(1)
    outs = []
    for j in range(HP):
        sl = slice(j * DH, (j + 1) * DH)
        s = _causal_scores(q_ref[:, sl], k_ref[:, sl], i)
        m = jnp.max(s, axis=-1, keepdims=True)
        p = jnp.exp(s - m)
        l = jnp.sum(p, axis=-1, keepdims=True)
        outs.append(jnp.dot(p, v_ref[:, sl],
                            preferred_element_type=jnp.float32) / l)
    o_ref[...] = jnp.concatenate(outs, axis=-1)


def _attn_local(q, k, v):
    head_row = pl.BlockSpec((QB, HP * DH), lambda h, i: (i, h))
    head_full = pl.BlockSpec((S, HP * DH), lambda h, i: (0, h))
    return pl.pallas_call(
        _attn_kernel,
        grid=(H // HP, NQ),
        in_specs=[head_row, head_full, head_full],
        out_specs=head_row,
        out_shape=jax.ShapeDtypeStruct((S, D), jnp.float32),
    )(q, k, v)


def _sim_kernel(q_ref, mk_ref, sim_ref):
    sims = []
    for j in range(HP):
        sl = slice(j * DH, (j + 1) * DH)
        sim = jax.lax.dot_general(
            q_ref[:, sl], mk_ref[:, sl], (((1,), (1,)), ((), ())),
            preferred_element_type=jnp.float32) * SCALE
        mu = jnp.mean(sim, axis=-1, keepdims=True)
        var = jnp.maximum(jnp.mean(sim * sim, axis=-1, keepdims=True) - mu * mu,
                          0.0)
        tau = mu + 2.0 * jnp.sqrt(var)
        tau16 = jnp.broadcast_to(tau, (QB, 16))
        sims.append(jnp.concatenate([tau16, sim], axis=-1))
    sim_ref[...] = jnp.stack(sims, axis=0)


def _sim_mem(q, mk2):
    return pl.pallas_call(
        _sim_kernel,
        grid=(H // HP, NQ),
        in_specs=[pl.BlockSpec((QB, HP * DH), lambda h, i: (i, h)),
                  pl.BlockSpec((M, HP * DH), lambda h, i: (0, h))],
        out_specs=pl.BlockSpec((HP, QB, MW), lambda h, i: (h, i, 0)),
        out_shape=jax.ShapeDtypeStruct((H, S, MW), jnp.float32),
    )(q, mk2)


def _f2key(v):
    i = plsc.bitcast(v, jnp.int32)
    return jnp.where(i < 0, i ^ MASK31, i)


def _key2f(kk):
    return plsc.bitcast(jnp.where(kk < 0, kk ^ MASK31, kk), jnp.float32)


def _popcnt(msk):
    return plsc.all_reduce_population_count(msk)[0]


def _count_ge(loader, nv, t_vec):
    def cb(j, acc):
        return acc + (loader(j) >= t_vec).astype(jnp.int32)
    acc = lax.fori_loop(0, nv, cb, jnp.zeros((16,), jnp.int32))
    return jnp.sum(acc)


def _bisect(loader, nv, lo0, hi0):
    def bb(_, lohi):
        lo, hi = lohi
        mid = lo + lax.shift_right_logical(hi - lo, 1)
        c = _count_ge(loader, nv, mid)
        take = c >= K
        return jnp.where(take, mid, lo), jnp.where(take, hi, mid)
    lo, _ = lax.fori_loop(0, 32, bb, (lo0, hi0))
    return lo


def _extract(loader, idx_loader, nv, tstar, otv, oti, obase):
    def ex_strict(j, po):
        kj = loader(j)
        m = kj > tstar
        cnt = _popcnt(m)

        @pl.when(cnt > 0)
        def _():
            plsc.store_compressed(otv.at[pl.ds(obase + po, 16)], _key2f(kj),
                                  mask=m)
            plsc.store_compressed(oti.at[pl.ds(obase + po, 16)], idx_loader(j),
                                  mask=m)

        return po + cnt

    po = lax.fori_loop(0, nv, ex_strict, jnp.int32(0))

    def ex_tie(j, po):
        kj = loader(j)
        m = kj == tstar
        cum = plsc.cumsum(m.astype(jnp.int32))
        keep = jnp.logical_and(m, cum <= (K - po))
        cnt = _popcnt(keep)

        @pl.when(cnt > 0)
        def _():
            plsc.store_compressed(otv.at[pl.ds(obase + po, 16)], _key2f(kj),
                                  mask=keep)
            plsc.store_compressed(oti.at[pl.ds(obase + po, 16)], idx_loader(j),
                                  mask=keep)

        return po + cnt

    lax.fori_loop(0, nv, ex_tie, po)


def _topk_sc(simt2, mv2):
    mesh = plsc.VectorSubcoreMesh(core_axis_name="c", subcore_axis_name="s")
    BLK = RB * MW
    GN = RB * K

    @functools.partial(
        pl.kernel, mesh=mesh,
        compiler_params=pltpu.CompilerParams(needs_layout_passes=False,
                                             use_tc_tiling_on_sc=False),
        out_type=[jax.ShapeDtypeStruct((NR * K,), jnp.float32),
                  jax.ShapeDtypeStruct((NR * K,), jnp.int32),
                  jax.ShapeDtypeStruct((NR * K, DH), jnp.float32)],
        scratch_types=[
            pltpu.VMEM((BLK,), jnp.float32),
            pltpu.VMEM((BLK,), jnp.float32),
            pltpu.VMEM((CAP + 16,), jnp.float32),
            pltpu.VMEM((CAP + 16,), jnp.int32),
            pltpu.VMEM((RB * K + 16,), jnp.float32),
            pltpu.VMEM((RB * K + 16,), jnp.int32),
            pltpu.VMEM((RB * K + 16,), jnp.float32),
            pltpu.VMEM((RB * K + 16,), jnp.int32),
            pltpu.VMEM((GN,), jnp.int32),
            pltpu.VMEM((GN,), jnp.int32),
            pltpu.VMEM((GN, DH), jnp.float32),
            pltpu.VMEM((GN, DH), jnp.float32),
            pltpu.SemaphoreType.DMA,
            pltpu.SemaphoreType.DMA,
            pltpu.SemaphoreType.DMA,
            pltpu.SemaphoreType.DMA,
            pltpu.SemaphoreType.DMA,
            pltpu.SemaphoreType.DMA,
            pltpu.SemaphoreType.DMA,
            pltpu.SemaphoreType.DMA,
        ],
    )
    def tk(simt_hbm, mv_hbm, tv_hbm, ti_hbm, rv_hbm, buf0, buf1, cval, cidx,
           otv0, oti0, otv1, oti1, gix0, gix1, grv0, grv1,
           dsem0, dsem1, osem0, osem1, gsem0, gsem1, rsem0, rsem1):
        cid = lax.axis_index("c")
        sid = lax.axis_index("s")
        wid = sid * 2 + cid
        base = wid * RPW
        hh = lax.div(wid, 2)

        def in_slice(jb):
            return simt_hbm.at[pl.ds((base + jb * RB) * MW, BLK)]

        def out_slices(jb):
            sl = pl.ds((base + jb * RB) * K, RB * K)
            return tv_hbm.at[sl], ti_hbm.at[sl]

        def rv_slice(jb):
            return rv_hbm.at[pl.ds((base + jb * RB) * K, GN), :]

        def gather_pair(gix, grv, gsem):
            pltpu.async_copy(mv_hbm.at[gix.at[pl.ds(0, 128)]],
                             grv.at[pl.ds(0, 128), :], gsem)
            pltpu.async_copy(mv_hbm.at[gix.at[pl.ds(128, 128)]],
                             grv.at[pl.ds(128, 128), :], gsem)

        def gather_wait(gix, grv, gsem):
            pltpu.make_async_copy(mv_hbm.at[gix.at[pl.ds(0, 128)]],
                                  grv.at[pl.ds(0, 128), :], gsem).wait()
            pltpu.make_async_copy(mv_hbm.at[gix.at[pl.ds(128, 128)]],
                                  grv.at[pl.ds(128, 128), :], gsem).wait()

        def process_row(buf, r, otv, oti):
            rbase = r * MW
            obase = r * K
            tauv = buf[pl.ds(rbase, 16)]

            def ap_body(j, pos):
                v = buf[pl.ds(rbase + 16 + j * 16, 16)]
                msk = v > tauv
                cnt = _popcnt(msk)

                @pl.when(cnt > 0)
                def _():
                    iv = lax.iota(jnp.int32, 16) + j * 16
                    plsc.store_compressed(cval.at[pl.ds(pos, 16)], v, mask=msk)
                    plsc.store_compressed(cidx.at[pl.ds(pos, 16)], iv, mask=msk)

                return pos + cnt

            pos = lax.fori_loop(0, M // 16, ap_body, jnp.int32(0))
            cval[pl.ds(pos, 16)] = jnp.full((16,), -jnp.inf, jnp.float32)

            hi0 = jnp.full((16,), KEY_INF, jnp.int32)

            @pl.when(pos >= K)
            def _():
                nv = (pos + 15) // 16
                loader = lambda j: _f2key(cval[pl.ds(j * 16, 16)])
                idx_loader = lambda j: cidx[pl.ds(j * 16, 16)]
                tstar = _bisect(loader, nv, _f2key(tauv), hi0)
                _extract(loader, idx_loader, nv, tstar, otv, oti, obase)

            @pl.when(pos < K)
            def _():
                loader = lambda j: _f2key(buf[pl.ds(rbase + 16 + j * 16, 16)])
                idx_loader = lambda j: lax.iota(jnp.int32, 16) + j * 16
                lo0 = jnp.full((16,), -(2 ** 31), jnp.int32)
                tstar = _bisect(loader, M // 16, lo0, hi0)
                _extract(loader, idx_loader, M // 16, tstar, otv, oti, obase)

        def do_block(jb, buf, dsem, nbuf, ndsem, otv, oti, osem,
                     gix, grv, gsem, rsem):
            pltpu.make_async_copy(in_slice(jb), buf, dsem).wait()

            @pl.when(jb + 1 < NB)
            def _():
                pltpu.async_copy(in_slice(jb + 1), nbuf, ndsem)

            @pl.when(jb >= 2)
            def _():
                gather_wait(gix, grv, gsem)
                pltpu.async_copy(grv, rv_slice(jb - 2), rsem)
                tvs, tis = out_slices(jb - 2)
                pltpu.make_async_copy(otv.at[pl.ds(0, RB * K)], tvs, osem).wait()
                pltpu.make_async_copy(oti.at[pl.ds(0, RB * K)], tis, osem).wait()

            def row_body(r, c):
                process_row(buf, r, otv, oti)
                return c

            lax.fori_loop(0, RB, row_body, jnp.int32(0))

            def gx_body(i, c):
                gix[pl.ds(i * 16, 16)] = oti[pl.ds(i * 16, 16)] * H + hh
                return c

            lax.fori_loop(0, GN // 16, gx_body, jnp.int32(0))

            @pl.when(jb >= 2)
            def _():
                pltpu.make_async_copy(grv, rv_slice(jb - 2), rsem).wait()

            gather_pair(gix, grv, gsem)

            tvs, tis = out_slices(jb)
            pltpu.async_copy(otv.at[pl.ds(0, RB * K)], tvs, osem)
            pltpu.async_copy(oti.at[pl.ds(0, RB * K)], tis, osem)

        pltpu.async_copy(in_slice(0), buf0, dsem0)

        def block_body(jb, c):
            par = lax.rem(jb, 2)

            @pl.when(par == 0)
            def _():
                do_block(jb, buf0, dsem0, buf1, dsem1, otv0, oti0, osem0,
                         gix0, grv0, gsem0, rsem0)

            @pl.when(par == 1)
            def _():
                do_block(jb, buf1, dsem1, buf0, dsem0, otv1, oti1, osem1,
                         gix1, grv1, gsem1, rsem1)

            return c

        lax.fori_loop(0, NB, block_body, jnp.int32(0))

        tvs, tis = out_slices(NB - 2)
        pltpu.make_async_copy(otv0.at[pl.ds(0, RB * K)], tvs, osem0).wait()
        pltpu.make_async_copy(oti0.at[pl.ds(0, RB * K)], tis, osem0).wait()
        tvs, tis = out_slices(NB - 1)
        pltpu.make_async_copy(otv1.at[pl.ds(0, RB * K)], tvs, osem1).wait()
        pltpu.make_async_copy(oti1.at[pl.ds(0, RB * K)], tis, osem1).wait()
        gather_wait(gix0, grv0, gsem0)
        pltpu.sync_copy(grv0, rv_slice(NB - 2))
        gather_wait(gix1, grv1, gsem1)
        pltpu.sync_copy(grv1, rv_slice(NB - 1))

    return tk(simt2, mv2)


def _attn_mem_kernel(q_ref, k_ref, v_ref, tv_ref, rv_ref, o_ref):
    i = pl.program_id(1)
    outs = []
    for j in range(HP):
        sl = slice(j * DH, (j + 1) * DH)
        s = _causal_scores(q_ref[:, sl], k_ref[:, sl], i)
        tv = tv_ref[j]
        m = jnp.maximum(jnp.max(s, axis=-1, keepdims=True),
                        jnp.max(tv, axis=-1, keepdims=True))
        p = jnp.exp(s - m)
        w = jnp.exp(tv - m)
        l = (jnp.sum(p, axis=-1, keepdims=True)
             + jnp.sum(w, axis=-1, keepdims=True))
        acc = jnp.dot(p, v_ref[:, sl], preferred_element_type=jnp.float32)
        for kk in range(K):
            acc = acc + w[:, kk:kk + 1] * rv_ref[j, :, kk, :]
        outs.append(acc / l)
    o_ref[...] = jnp.concatenate(outs, axis=-1)


def _attn_with_mem(q, k, v, mem_scores, retrieved):
    head_row = pl.BlockSpec((QB, HP * DH), lambda h, i: (i, h))
    head_full = pl.BlockSpec((S, HP * DH), lambda h, i: (0, h))
    return pl.pallas_call(
        _attn_mem_kernel,
        grid=(H // HP, NQ),
        in_specs=[head_row, head_full, head_full,
                  pl.BlockSpec((HP, QB, K), lambda h, i: (h, i, 0)),
                  pl.BlockSpec((HP, QB, K, DH), lambda h, i: (h, i, 0, 0))],
        out_specs=head_row,
        out_shape=jax.ShapeDtypeStruct((S, D), jnp.float32),
    )(q, k, v, mem_scores, retrieved)


def _proj_ffn_kernel(a_ref, x_ref, wo_ref, g2_ref, w1_ref, w2_ref, o_ref):
    xx = x_ref[...] + jnp.dot(a_ref[...], wo_ref[...],
                              preferred_element_type=jnp.float32)
    mu = jnp.mean(xx, axis=-1, keepdims=True)
    var = jnp.mean((xx - mu) ** 2, axis=-1, keepdims=True)
    h2 = (xx - mu) * jax.lax.rsqrt(var + 1e-5) * g2_ref[...]
    t = jax.nn.gelu(jnp.dot(h2, w1_ref[...], preferred_element_type=jnp.float32))
    o_ref[...] = xx + jnp.dot(t, w2_ref[...], preferred_element_type=jnp.float32)


def _proj_ffn(attn_out, x, wo, g2, w1, w2):
    row = pl.BlockSpec((QB, D), lambda i: (i, 0))
    return pl.pallas_call(
        _proj_ffn_kernel,
        grid=(NQ,),
        in_specs=[row, row,
                  pl.BlockSpec((D, D), lambda i: (0, 0)),
                  pl.BlockSpec((1, D), lambda i: (0, 0)),
                  pl.BlockSpec((D, FF), lambda i: (0, 0)),
                  pl.BlockSpec((FF, D), lambda i: (0, 0))],
        out_specs=row,
        out_shape=jax.ShapeDtypeStruct((S, D), jnp.float32),
    )(attn_out, x, wo, g2, w1, w2)


def kernel(x, batch_indices, mem_k, mem_v, ln1, wq, wk, wv, wo, gate, ln2, w1, w2):
    xx = x[0]
    mk2 = mem_k[0].reshape(M, D)

    for l in range(L):
        q, k, v = _qkv(xx, ln1[l][None], wq[l], wk[l], wv[l])
        if l == 1:
            simt = _sim_mem(q, mk2)
            tv_f, ti_f, rv_f = _topk_sc(simt.reshape(NR * MW),
                                        mem_v[0].reshape(M * H, DH))
            top_vals = tv_f.reshape(H, S, K)
            retrieved = rv_f.reshape(H, S, K, DH)
            mem_scores = top_vals + gate[l][:, None, None]
            attn_out = _attn_with_mem(q, k, v, mem_scores, retrieved)
        else:
            attn_out = _attn_local(q, k, v)
        xx = _proj_ffn(attn_out, xx, wo[l], ln2[l][None], w1[l], w2[l])
    return xx[None]


# speedup vs baseline: 1.2445x; 1.2445x over previous
import functools

import jax
import jax.numpy as jnp
from jax import lax
from jax.experimental import pallas as pl
from jax.experimental.pallas import tpu as pltpu
from jax.experimental.pallas import tpu_sc as plsc

B, S, D, H, L = 1, 2048, 1024, 16, 2
DH = D // H
M, K = 4096, 32
FF = 4 * D
SCALE = DH ** -0.5
QB = 256
NQ = S // QB
HP = 2

NR = H * S
MW = M + 16
NWORK = 32
RPW = NR // NWORK
RB = 8
NB = RPW // RB
CAP = 1088
OW = K + 16
KEY_INF = 0x7F800000
MASK31 = 0x7FFFFFFF


def _qkv_kernel(x_ref, g_ref, wq_ref, wk_ref, wv_ref, q_ref, k_ref, v_ref):
    x = x_ref[...]
    mu = jnp.mean(x, axis=-1, keepdims=True)
    var = jnp.mean((x - mu) ** 2, axis=-1, keepdims=True)
    h = (x - mu) * jax.lax.rsqrt(var + 1e-5) * g_ref[...]
    q_ref[...] = jnp.dot(h, wq_ref[...], preferred_element_type=jnp.float32)
    k_ref[...] = jnp.dot(h, wk_ref[...], preferred_element_type=jnp.float32)
    v_ref[...] = jnp.dot(h, wv_ref[...], preferred_element_type=jnp.float32)


def _qkv(x, g, wq, wk, wv):
    shp = jax.ShapeDtypeStruct((S, D), jnp.float32)
    full = pl.BlockSpec((D, D), lambda i: (0, 0))
    row = pl.BlockSpec((QB, D), lambda i: (i, 0))
    return pl.pallas_call(
        _qkv_kernel,
        grid=(NQ,),
        in_specs=[row, pl.BlockSpec((1, D), lambda i: (0, 0)), full, full, full],
        out_specs=[row, row, row],
        out_shape=[shp, shp, shp],
    )(x, g, wq, wk, wv)


def _causal_scores(q, k_all, i):
    s = jax.lax.dot_general(q, k_all, (((1,), (1,)), ((), ())),
                            preferred_element_type=jnp.float32) * SCALE
    rows = i * QB + jax.lax.broadcasted_iota(jnp.int32, (QB, S), 0)
    cols = jax.lax.broadcasted_iota(jnp.int32, (QB, S), 1)
    return jnp.where(cols <= rows, s, -1e9)


def _attn_kernel(q_ref, k_ref, v_ref, o_ref):
    i = pl.program_id(1)
    outs = []
    for j in range(HP):
        sl = slice(j * DH, (j + 1) * DH)
        s = _causal_scores(q_ref[:, sl], k_ref[:, sl], i)
        m = jnp.max(s, axis=-1, keepdims=True)
        p = jnp.exp(s - m)
        l = jnp.sum(p, axis=-1, keepdims=True)
        outs.append(jnp.dot(p, v_ref[:, sl],
                            preferred_element_type=jnp.float32) / l)
    o_ref[...] = jnp.concatenate(outs, axis=-1)


def _attn_local(q, k, v):
    head_row = pl.BlockSpec((QB, HP * DH), lambda h, i: (i, h))
    head_full = pl.BlockSpec((S, HP * DH), lambda h, i: (0, h))
    return pl.pallas_call(
        _attn_kernel,
        grid=(H // HP, NQ),
        in_specs=[head_row, head_full, head_full],
        out_specs=head_row,
        out_shape=jax.ShapeDtypeStruct((S, D), jnp.float32),
    )(q, k, v)


def _sim_kernel(q_ref, mk_ref, sim_ref):
    sims = []
    for j in range(HP):
        sl = slice(j * DH, (j + 1) * DH)
        sim = jax.lax.dot_general(
            q_ref[:, sl], mk_ref[:, sl], (((1,), (1,)), ((), ())),
            preferred_element_type=jnp.float32) * SCALE
        mu = jnp.mean(sim, axis=-1, keepdims=True)
        var = jnp.maximum(jnp.mean(sim * sim, axis=-1, keepdims=True) - mu * mu,
                          0.0)
        tau = mu + 2.0 * jnp.sqrt(var)
        tau16 = jnp.broadcast_to(tau, (QB, 16))
        sims.append(jnp.concatenate([tau16, sim], axis=-1))
    sim_ref[...] = jnp.stack(sims, axis=0)


def _sim_mem(q, mk2):
    return pl.pallas_call(
        _sim_kernel,
        grid=(H // HP, NQ),
        in_specs=[pl.BlockSpec((QB, HP * DH), lambda h, i: (i, h)),
                  pl.BlockSpec((M, HP * DH), lambda h, i: (0, h))],
        out_specs=pl.BlockSpec((HP, QB, MW), lambda h, i: (h, i, 0)),
        out_shape=jax.ShapeDtypeStruct((H, S, MW), jnp.float32),
    )(q, mk2)


def _f2key(v):
    i = plsc.bitcast(v, jnp.int32)
    return jnp.where(i < 0, i ^ MASK31, i)


def _key2f(kk):
    return plsc.bitcast(jnp.where(kk < 0, kk ^ MASK31, kk), jnp.float32)


def _popcnt(msk):
    return plsc.all_reduce_population_count(msk)[0]


def _count_ge(loader, nv, t_vec):
    def cb(j, acc):
        return acc + (loader(j) >= t_vec).astype(jnp.int32)
    acc = lax.fori_loop(0, nv, cb, jnp.zeros((16,), jnp.int32))
    return jnp.sum(acc)


def _bisect(loader, nv, lo0, hi0):
    def bb(_, lohi):
        lo, hi = lohi
        mid = lo + lax.shift_right_logical(hi - lo, 1)
        c = _count_ge(loader, nv, mid)
        take = c >= K
        return jnp.where(take, mid, lo), jnp.where(take, hi, mid)
    lo, _ = lax.fori_loop(0, 32, bb, (lo0, hi0))
    return lo


def _extract(loader, idx_loader, nv, tstar, otv, oti, obase):
    def ex_strict(j, po):
        kj = loader(j)
        m = kj > tstar
        plsc.store_compressed(otv.at[pl.ds(obase + po, 16)], _key2f(kj),
                              mask=m)
        plsc.store_compressed(oti.at[pl.ds(obase + po, 16)], idx_loader(j),
                              mask=m)
        return po + _popcnt(m)

    po = lax.fori_loop(0, nv, ex_strict, jnp.int32(0))

    def ex_tie(j, po):
        kj = loader(j)
        m = kj == tstar
        cum = plsc.cumsum(m.astype(jnp.int32))
        keep = jnp.logical_and(m, cum <= (K - po))
        plsc.store_compressed(otv.at[pl.ds(obase + po, 16)], _key2f(kj),
                              mask=keep)
        plsc.store_compressed(oti.at[pl.ds(obase + po, 16)], idx_loader(j),
                              mask=keep)
        return po + _popcnt(keep)

    lax.fori_loop(0, nv, ex_tie, po)


def _topk_sc(simt2, mv2):
    mesh = plsc.VectorSubcoreMesh(core_axis_name="c", subcore_axis_name="s")
    BLK = RB * MW
    GN = RB * K

    @functools.partial(
        pl.kernel, mesh=mesh,
        compiler_params=pltpu.CompilerParams(needs_layout_passes=False,
                                             use_tc_tiling_on_sc=False),
        out_type=[jax.ShapeDtypeStruct((NR * K,), jnp.float32),
                  jax.ShapeDtypeStruct((NR * K,), jnp.int32),
                  jax.ShapeDtypeStruct((NR * K, DH), jnp.float32)],
        scratch_types=[
            pltpu.VMEM((BLK,), jnp.float32),
            pltpu.VMEM((BLK,), jnp.float32),
            pltpu.VMEM((CAP + 16,), jnp.float32),
            pltpu.VMEM((CAP + 16,), jnp.int32),
            pltpu.VMEM((RB * K + 16,), jnp.float32),
            pltpu.VMEM((RB * K + 16,), jnp.int32),
            pltpu.VMEM((RB * K + 16,), jnp.float32),
            pltpu.VMEM((RB * K + 16,), jnp.int32),
            pltpu.VMEM((GN,), jnp.int32),
            pltpu.VMEM((GN,), jnp.int32),
            pltpu.VMEM((GN, DH), jnp.float32),
            pltpu.VMEM((GN, DH), jnp.float32),
            pltpu.SemaphoreType.DMA,
            pltpu.SemaphoreType.DMA,
            pltpu.SemaphoreType.DMA,
            pltpu.SemaphoreType.DMA,
            pltpu.SemaphoreType.DMA,
            pltpu.SemaphoreType.DMA,
            pltpu.SemaphoreType.DMA,
            pltpu.SemaphoreType.DMA,
        ],
    )
    def tk(simt_hbm, mv_hbm, tv_hbm, ti_hbm, rv_hbm, buf0, buf1, cval, cidx,
           otv0, oti0, otv1, oti1, gix0, gix1, grv0, grv1,
           dsem0, dsem1, osem0, osem1, gsem0, gsem1, rsem0, rsem1):
        cid = lax.axis_index("c")
        sid = lax.axis_index("s")
        wid = sid * 2 + cid
        base = wid * RPW
        hh = lax.div(wid, 2)

        def in_slice(jb):
            return simt_hbm.at[pl.ds((base + jb * RB) * MW, BLK)]

        def out_slices(jb):
            sl = pl.ds((base + jb * RB) * K, RB * K)
            return tv_hbm.at[sl], ti_hbm.at[sl]

        def rv_slice(jb):
            return rv_hbm.at[pl.ds((base + jb * RB) * K, GN), :]

        def gather_pair(gix, grv, gsem):
            pltpu.async_copy(mv_hbm.at[gix.at[pl.ds(0, 128)]],
                             grv.at[pl.ds(0, 128), :], gsem)
            pltpu.async_copy(mv_hbm.at[gix.at[pl.ds(128, 128)]],
                             grv.at[pl.ds(128, 128), :], gsem)

        def gather_wait(gix, grv, gsem):
            pltpu.make_async_copy(mv_hbm.at[gix.at[pl.ds(0, 128)]],
                                  grv.at[pl.ds(0, 128), :], gsem).wait()
            pltpu.make_async_copy(mv_hbm.at[gix.at[pl.ds(128, 128)]],
                                  grv.at[pl.ds(128, 128), :], gsem).wait()

        def process_row(buf, r, otv, oti):
            rbase = r * MW
            obase = r * K
            tauv = buf[pl.ds(rbase, 16)]

            def ap_body(j, pos):
                v = buf[pl.ds(rbase + 16 + j * 16, 16)]
                msk = v > tauv
                iv = lax.iota(jnp.int32, 16) + j * 16
                plsc.store_compressed(cval.at[pl.ds(pos, 16)], v, mask=msk)
                plsc.store_compressed(cidx.at[pl.ds(pos, 16)], iv, mask=msk)
                return pos + _popcnt(msk)

            pos = lax.fori_loop(0, M // 16, ap_body, jnp.int32(0))
            cval[pl.ds(pos, 16)] = jnp.full((16,), -jnp.inf, jnp.float32)

            hi0 = jnp.full((16,), KEY_INF, jnp.int32)

            @pl.when(pos >= K)
            def _():
                nv = (pos + 15) // 16
                loader = lambda j: _f2key(cval[pl.ds(j * 16, 16)])
                idx_loader = lambda j: cidx[pl.ds(j * 16, 16)]
                tstar = _bisect(loader, nv, _f2key(tauv), hi0)
                _extract(loader, idx_loader, nv, tstar, otv, oti, obase)

            @pl.when(pos < K)
            def _():
                loader = lambda j: _f2key(buf[pl.ds(rbase + 16 + j * 16, 16)])
                idx_loader = lambda j: lax.iota(jnp.int32, 16) + j * 16
                lo0 = jnp.full((16,), -(2 ** 31), jnp.int32)
                tstar = _bisect(loader, M // 16, lo0, hi0)
                _extract(loader, idx_loader, M // 16, tstar, otv, oti, obase)

        def do_block(jb, buf, dsem, nbuf, ndsem, otv, oti, osem,
                     gix, grv, gsem, rsem):
            pltpu.make_async_copy(in_slice(jb), buf, dsem).wait()

            @pl.when(jb + 1 < NB)
            def _():
                pltpu.async_copy(in_slice(jb + 1), nbuf, ndsem)

            @pl.when(jb >= 2)
            def _():
                gather_wait(gix, grv, gsem)
                pltpu.async_copy(grv, rv_slice(jb - 2), rsem)
                tvs, tis = out_slices(jb - 2)
                pltpu.make_async_copy(otv.at[pl.ds(0, RB * K)], tvs, osem).wait()
                pltpu.make_async_copy(oti.at[pl.ds(0, RB * K)], tis, osem).wait()

            def row_body(r, c):
                process_row(buf, r, otv, oti)
                return c

            lax.fori_loop(0, RB, row_body, jnp.int32(0))

            def gx_body(i, c):
                gix[pl.ds(i * 16, 16)] = oti[pl.ds(i * 16, 16)] * H + hh
                return c

            lax.fori_loop(0, GN // 16, gx_body, jnp.int32(0))

            @pl.when(jb >= 2)
            def _():
                pltpu.make_async_copy(grv, rv_slice(jb - 2), rsem).wait()

            gather_pair(gix, grv, gsem)

            tvs, tis = out_slices(jb)
            pltpu.async_copy(otv.at[pl.ds(0, RB * K)], tvs, osem)
            pltpu.async_copy(oti.at[pl.ds(0, RB * K)], tis, osem)

        pltpu.async_copy(in_slice(0), buf0, dsem0)

        def block_body(jb, c):
            par = lax.rem(jb, 2)

            @pl.when(par == 0)
            def _():
                do_block(jb, buf0, dsem0, buf1, dsem1, otv0, oti0, osem0,
                         gix0, grv0, gsem0, rsem0)

            @pl.when(par == 1)
            def _():
                do_block(jb, buf1, dsem1, buf0, dsem0, otv1, oti1, osem1,
                         gix1, grv1, gsem1, rsem1)

            return c

        lax.fori_loop(0, NB, block_body, jnp.int32(0))

        tvs, tis = out_slices(NB - 2)
        pltpu.make_async_copy(otv0.at[pl.ds(0, RB * K)], tvs, osem0).wait()
        pltpu.make_async_copy(oti0.at[pl.ds(0, RB * K)], tis, osem0).wait()
        tvs, tis = out_slices(NB - 1)
        pltpu.make_async_copy(otv1.at[pl.ds(0, RB * K)], tvs, osem1).wait()
        pltpu.make_async_copy(oti1.at[pl.ds(0, RB * K)], tis, osem1).wait()
        gather_wait(gix0, grv0, gsem0)
        pltpu.sync_copy(grv0, rv_slice(NB - 2))
        gather_wait(gix1, grv1, gsem1)
        pltpu.sync_copy(grv1, rv_slice(NB - 1))

    return tk(simt2, mv2)


def _attn_mem_kernel(q_ref, k_ref, v_ref, tv_ref, rv_ref, o_ref):
    i = pl.program_id(1)
    outs = []
    for j in range(HP):
        sl = slice(j * DH, (j + 1) * DH)
        s = _causal_scores(q_ref[:, sl], k_ref[:, sl], i)
        tv = tv_ref[j]
        m = jnp.maximum(jnp.max(s, axis=-1, keepdims=True),
                        jnp.max(tv, axis=-1, keepdims=True))
        p = jnp.exp(s - m)
        w = jnp.exp(tv - m)
        l = (jnp.sum(p, axis=-1, keepdims=True)
             + jnp.sum(w, axis=-1, keepdims=True))
        acc = jnp.dot(p, v_ref[:, sl], preferred_element_type=jnp.float32)
        for kk in range(K):
            acc = acc + w[:, kk:kk + 1] * rv_ref[j, :, kk, :]
        outs.append(acc / l)
    o_ref[...] = jnp.concatenate(outs, axis=-1)


def _attn_with_mem(q, k, v, mem_scores, retrieved):
    head_row = pl.BlockSpec((QB, HP * DH), lambda h, i: (i, h))
    head_full = pl.BlockSpec((S, HP * DH), lambda h, i: (0, h))
    return pl.pallas_call(
        _attn_mem_kernel,
        grid=(H // HP, NQ),
        in_specs=[head_row, head_full, head_full,
                  pl.BlockSpec((HP, QB, K), lambda h, i: (h, i, 0)),
                  pl.BlockSpec((HP, QB, K, DH), lambda h, i: (h, i, 0, 0))],
        out_specs=head_row,
        out_shape=jax.ShapeDtypeStruct((S, D), jnp.float32),
    )(q, k, v, mem_scores, retrieved)


def _proj_ffn_kernel(a_ref, x_ref, wo_ref, g2_ref, w1_ref, w2_ref, o_ref):
    xx = x_ref[...] + jnp.dot(a_ref[...], wo_ref[...],
                              preferred_element_type=jnp.float32)
    mu = jnp.mean(xx, axis=-1, keepdims=True)
    var = jnp.mean((xx - mu) ** 2, axis=-1, keepdims=True)
    h2 = (xx - mu) * jax.lax.rsqrt(var + 1e-5) * g2_ref[...]
    t = jax.nn.gelu(jnp.dot(h2, w1_ref[...], preferred_element_type=jnp.float32))
    o_ref[...] = xx + jnp.dot(t, w2_ref[...], preferred_element_type=jnp.float32)


def _proj_ffn(attn_out, x, wo, g2, w1, w2):
    row = pl.BlockSpec((QB, D), lambda i: (i, 0))
    return pl.pallas_call(
        _proj_ffn_kernel,
        grid=(NQ,),
        in_specs=[row, row,
                  pl.BlockSpec((D, D), lambda i: (0, 0)),
                  pl.BlockSpec((1, D), lambda i: (0, 0)),
                  pl.BlockSpec((D, FF), lambda i: (0, 0)),
                  pl.BlockSpec((FF, D), lambda i: (0, 0))],
        out_specs=row,
        out_shape=jax.ShapeDtypeStruct((S, D), jnp.float32),
    )(attn_out, x, wo, g2, w1, w2)


def kernel(x, batch_indices, mem_k, mem_v, ln1, wq, wk, wv, wo, gate, ln2, w1, w2):
    xx = x[0]
    mk2 = mem_k[0].reshape(M, D)

    for l in range(L):
        q, k, v = _qkv(xx, ln1[l][None], wq[l], wk[l], wv[l])
        if l == 1:
            simt = _sim_mem(q, mk2)
            tv_f, ti_f, rv_f = _topk_sc(simt.reshape(NR * MW),
                                        mem_v[0].reshape(M * H, DH))
            top_vals = tv_f.reshape(H, S, K)
            retrieved = rv_f.reshape(H, S, K, DH)
            mem_scores = top_vals + gate[l][:, None, None]
            attn_out = _attn_with_mem(q, k, v, mem_scores, retrieved)
        else:
            attn_out = _attn_local(q, k, v)
        xx = _proj_ffn(attn_out, xx, wo[l], ln2[l][None], w1[l], w2[l])
    return xx[None]


# speedup vs baseline: 1.3079x; 1.0509x over previous
import functools

import jax
import jax.numpy as jnp
from jax import lax
from jax.experimental import pallas as pl
from jax.experimental.pallas import tpu as pltpu
from jax.experimental.pallas import tpu_sc as plsc

B, S, D, H, L = 1, 2048, 1024, 16, 2
DH = D // H
M, K = 4096, 32
FF = 4 * D
SCALE = DH ** -0.5
QB = 256
NQ = S // QB
HP = 2

NR = H * S
MW = M + 16
NWORK = 32
RPW = NR // NWORK
RB = 8
NB = RPW // RB
CAP = 1088
OW = K + 16
KEY_INF = 0x7F800000
MASK31 = 0x7FFFFFFF


def _qkv_kernel(x_ref, g_ref, wq_ref, wk_ref, wv_ref, q_ref, k_ref, v_ref):
    x = x_ref[...]
    mu = jnp.mean(x, axis=-1, keepdims=True)
    var = jnp.mean((x - mu) ** 2, axis=-1, keepdims=True)
    h = (x - mu) * jax.lax.rsqrt(var + 1e-5) * g_ref[...]
    q_ref[...] = jnp.dot(h, wq_ref[...], preferred_element_type=jnp.float32)
    k_ref[...] = jnp.dot(h, wk_ref[...], preferred_element_type=jnp.float32)
    v_ref[...] = jnp.dot(h, wv_ref[...], preferred_element_type=jnp.float32)


def _qkv(x, g, wq, wk, wv):
    shp = jax.ShapeDtypeStruct((S, D), jnp.float32)
    full = pl.BlockSpec((D, D), lambda i: (0, 0))
    row = pl.BlockSpec((QB, D), lambda i: (i, 0))
    return pl.pallas_call(
        _qkv_kernel,
        grid=(NQ,),
        in_specs=[row, pl.BlockSpec((1, D), lambda i: (0, 0)), full, full, full],
        out_specs=[row, row, row],
        out_shape=[shp, shp, shp],
    )(x, g, wq, wk, wv)


def _causal_scores(q, k_all, i):
    s = jax.lax.dot_general(q, k_all, (((1,), (1,)), ((), ())),
                            preferred_element_type=jnp.float32) * SCALE
    rows = i * QB + jax.lax.broadcasted_iota(jnp.int32, (QB, S), 0)
    cols = jax.lax.broadcasted_iota(jnp.int32, (QB, S), 1)
    return jnp.where(cols <= rows, s, -1e9)


def _attn_kernel(q_ref, k_ref, v_ref, o_ref):
    i = pl.program_id(1)
    outs = []
    for j in range(HP):
        sl = slice(j * DH, (j + 1) * DH)
        s = _causal_scores(q_ref[:, sl], k_ref[:, sl], i)
        m = jnp.max(s, axis=-1, keepdims=True)
        p = jnp.exp(s - m)
        l = jnp.sum(p, axis=-1, keepdims=True)
        outs.append(jnp.dot(p, v_ref[:, sl],
                            preferred_element_type=jnp.float32) / l)
    o_ref[...] = jnp.concatenate(outs, axis=-1)


def _attn_local(q, k, v):
    head_row = pl.BlockSpec((QB, HP * DH), lambda h, i: (i, h))
    head_full = pl.BlockSpec((S, HP * DH), lambda h, i: (0, h))
    return pl.pallas_call(
        _attn_kernel,
        grid=(H // HP, NQ),
        in_specs=[head_row, head_full, head_full],
        out_specs=head_row,
        out_shape=jax.ShapeDtypeStruct((S, D), jnp.float32),
    )(q, k, v)


def _sim_kernel(q_ref, mk_ref, sim_ref):
    sims = []
    for j in range(HP):
        sl = slice(j * DH, (j + 1) * DH)
        sim = jax.lax.dot_general(
            q_ref[:, sl], mk_ref[:, sl], (((1,), (1,)), ((), ())),
            preferred_element_type=jnp.float32) * SCALE
        mu = jnp.mean(sim, axis=-1, keepdims=True)
        var = jnp.maximum(jnp.mean(sim * sim, axis=-1, keepdims=True) - mu * mu,
                          0.0)
        tau = mu + 2.0 * jnp.sqrt(var)
        tau16 = jnp.broadcast_to(tau, (QB, 16))
        sims.append(jnp.concatenate([tau16, sim], axis=-1))
    sim_ref[...] = jnp.stack(sims, axis=0)


def _sim_mem(q, mk2):
    return pl.pallas_call(
        _sim_kernel,
        grid=(H // HP, NQ),
        in_specs=[pl.BlockSpec((QB, HP * DH), lambda h, i: (i, h)),
                  pl.BlockSpec((M, HP * DH), lambda h, i: (0, h))],
        out_specs=pl.BlockSpec((HP, QB, MW), lambda h, i: (h, i, 0)),
        out_shape=jax.ShapeDtypeStruct((H, S, MW), jnp.float32),
    )(q, mk2)


def _f2key(v):
    i = plsc.bitcast(v, jnp.int32)
    return jnp.where(i < 0, i ^ MASK31, i)


def _key2f(kk):
    return plsc.bitcast(jnp.where(kk < 0, kk ^ MASK31, kk), jnp.float32)


def _popcnt(msk):
    return plsc.all_reduce_population_count(msk)[0]


def _count_ge(loader, nv, t_vec):
    def cb(j, acc):
        return acc + (loader(j) >= t_vec).astype(jnp.int32)
    acc = lax.fori_loop(0, nv, cb, jnp.zeros((16,), jnp.int32))
    return jnp.sum(acc)


def _bisect(loader, nv, lo0, hi0):
    def bb(_, lohi):
        lo, hi = lohi
        mid = lo + lax.shift_right_logical(hi - lo, 1)
        c = _count_ge(loader, nv, mid)
        take = c >= K
        return jnp.where(take, mid, lo), jnp.where(take, hi, mid)
    lo, _ = lax.fori_loop(0, 32, bb, (lo0, hi0))
    return lo


def _extract(loader, idx_loader, nv, tstar, otv, oti, obase):
    def ex_strict(j, po):
        kj = loader(j)
        m = kj > tstar
        plsc.store_compressed(otv.at[pl.ds(obase + po, 16)], _key2f(kj),
                              mask=m)
        plsc.store_compressed(oti.at[pl.ds(obase + po, 16)], idx_loader(j),
                              mask=m)
        return po + _popcnt(m)

    po = lax.fori_loop(0, nv, ex_strict, jnp.int32(0))

    def ex_tie(j, po):
        kj = loader(j)
        m = kj == tstar
        cum = plsc.cumsum(m.astype(jnp.int32))
        keep = jnp.logical_and(m, cum <= (K - po))
        plsc.store_compressed(otv.at[pl.ds(obase + po, 16)], _key2f(kj),
                              mask=keep)
        plsc.store_compressed(oti.at[pl.ds(obase + po, 16)], idx_loader(j),
                              mask=keep)
        return po + _popcnt(keep)

    lax.fori_loop(0, nv, ex_tie, po)


def _topk_sc(simt2, mv2):
    mesh = plsc.VectorSubcoreMesh(core_axis_name="c", subcore_axis_name="s")
    BLK = RB * MW
    GN = RB * K

    @functools.partial(
        pl.kernel, mesh=mesh,
        compiler_params=pltpu.CompilerParams(needs_layout_passes=False,
                                             use_tc_tiling_on_sc=False),
        out_type=[jax.ShapeDtypeStruct((NR * K,), jnp.float32),
                  jax.ShapeDtypeStruct((NR * K,), jnp.int32),
                  jax.ShapeDtypeStruct((NR * K, DH), jnp.float32)],
        scratch_types=[
            pltpu.VMEM((BLK,), jnp.float32),
            pltpu.VMEM((BLK,), jnp.float32),
            pltpu.VMEM((CAP + 16,), jnp.float32),
            pltpu.VMEM((CAP + 16,), jnp.int32),
            pltpu.VMEM((RB * K + 16,), jnp.float32),
            pltpu.VMEM((RB * K + 16,), jnp.int32),
            pltpu.VMEM((RB * K + 16,), jnp.float32),
            pltpu.VMEM((RB * K + 16,), jnp.int32),
            pltpu.VMEM((GN,), jnp.int32),
            pltpu.VMEM((GN,), jnp.int32),
            pltpu.VMEM((GN, DH), jnp.float32),
            pltpu.VMEM((GN, DH), jnp.float32),
            pltpu.SemaphoreType.DMA,
            pltpu.SemaphoreType.DMA,
            pltpu.SemaphoreType.DMA,
            pltpu.SemaphoreType.DMA,
            pltpu.SemaphoreType.DMA,
            pltpu.SemaphoreType.DMA,
            pltpu.SemaphoreType.DMA,
            pltpu.SemaphoreType.DMA,
        ],
    )
    def tk(simt_hbm, mv_hbm, tv_hbm, ti_hbm, rv_hbm, buf0, buf1, cval, cidx,
           otv0, oti0, otv1, oti1, gix0, gix1, grv0, grv1,
           dsem0, dsem1, osem0, osem1, gsem0, gsem1, rsem0, rsem1):
        cid = lax.axis_index("c")
        sid = lax.axis_index("s")
        wid = sid * 2 + cid
        base = wid * RPW
        hh = lax.div(wid, 2)

        def in_slice(jb):
            return simt_hbm.at[pl.ds((base + jb * RB) * MW, BLK)]

        def out_slices(jb):
            sl = pl.ds((base + jb * RB) * K, RB * K)
            return tv_hbm.at[sl], ti_hbm.at[sl]

        def rv_slice(jb):
            return rv_hbm.at[pl.ds((base + jb * RB) * K, GN), :]

        def gather_pair(gix, grv, gsem):
            pltpu.async_copy(mv_hbm.at[gix.at[pl.ds(0, 128)]],
                             grv.at[pl.ds(0, 128), :], gsem)
            pltpu.async_copy(mv_hbm.at[gix.at[pl.ds(128, 128)]],
                             grv.at[pl.ds(128, 128), :], gsem)

        def gather_wait(gix, grv, gsem):
            pltpu.make_async_copy(mv_hbm.at[gix.at[pl.ds(0, 128)]],
                                  grv.at[pl.ds(0, 128), :], gsem).wait()
            pltpu.make_async_copy(mv_hbm.at[gix.at[pl.ds(128, 128)]],
                                  grv.at[pl.ds(128, 128), :], gsem).wait()

        def process_row(buf, r, otv, oti):
            rbase = r * MW
            obase = r * K
            tauv = buf[pl.ds(rbase, 16)]

            def ap_body(j, pos):
                for u in range(4):
                    jj = j * 4 + u
                    v = buf[pl.ds(rbase + 16 + jj * 16, 16)]
                    msk = v > tauv
                    iv = lax.iota(jnp.int32, 16) + jj * 16
                    plsc.store_compressed(cval.at[pl.ds(pos, 16)], v, mask=msk)
                    plsc.store_compressed(cidx.at[pl.ds(pos, 16)], iv, mask=msk)
                    pos = pos + _popcnt(msk)
                return pos

            pos = lax.fori_loop(0, M // 64, ap_body, jnp.int32(0))
            cval[pl.ds(pos, 16)] = jnp.full((16,), -jnp.inf, jnp.float32)

            hi0 = jnp.full((16,), KEY_INF, jnp.int32)

            @pl.when(pos >= K)
            def _():
                nv = (pos + 15) // 16
                loader = lambda j: _f2key(cval[pl.ds(j * 16, 16)])
                idx_loader = lambda j: cidx[pl.ds(j * 16, 16)]
                tstar = _bisect(loader, nv, _f2key(tauv), hi0)
                _extract(loader, idx_loader, nv, tstar, otv, oti, obase)

            @pl.when(pos < K)
            def _():
                loader = lambda j: _f2key(buf[pl.ds(rbase + 16 + j * 16, 16)])
                idx_loader = lambda j: lax.iota(jnp.int32, 16) + j * 16
                lo0 = jnp.full((16,), -(2 ** 31), jnp.int32)
                tstar = _bisect(loader, M // 16, lo0, hi0)
                _extract(loader, idx_loader, M // 16, tstar, otv, oti, obase)

        def do_block(jb, buf, dsem, nbuf, ndsem, otv, oti, osem,
                     gix, grv, gsem, rsem):
            pltpu.make_async_copy(in_slice(jb), buf, dsem).wait()

            @pl.when(jb + 1 < NB)
            def _():
                pltpu.async_copy(in_slice(jb + 1), nbuf, ndsem)

            @pl.when(jb >= 2)
            def _():
                gather_wait(gix, grv, gsem)
                pltpu.async_copy(grv, rv_slice(jb - 2), rsem)
                tvs, tis = out_slices(jb - 2)
                pltpu.make_async_copy(otv.at[pl.ds(0, RB * K)], tvs, osem).wait()
                pltpu.make_async_copy(oti.at[pl.ds(0, RB * K)], tis, osem).wait()

            def row_body(r, c):
                process_row(buf, r, otv, oti)
                return c

            lax.fori_loop(0, RB, row_body, jnp.int32(0))

            def gx_body(i, c):
                gix[pl.ds(i * 16, 16)] = oti[pl.ds(i * 16, 16)] * H + hh
                return c

            lax.fori_loop(0, GN // 16, gx_body, jnp.int32(0))

            @pl.when(jb >= 2)
            def _():
                pltpu.make_async_copy(grv, rv_slice(jb - 2), rsem).wait()

            gather_pair(gix, grv, gsem)

            tvs, tis = out_slices(jb)
            pltpu.async_copy(otv.at[pl.ds(0, RB * K)], tvs, osem)
            pltpu.async_copy(oti.at[pl.ds(0, RB * K)], tis, osem)

        pltpu.async_copy(in_slice(0), buf0, dsem0)

        def block_body(jb, c):
            par = lax.rem(jb, 2)

            @pl.when(par == 0)
            def _():
                do_block(jb, buf0, dsem0, buf1, dsem1, otv0, oti0, osem0,
                         gix0, grv0, gsem0, rsem0)

            @pl.when(par == 1)
            def _():
                do_block(jb, buf1, dsem1, buf0, dsem0, otv1, oti1, osem1,
                         gix1, grv1, gsem1, rsem1)

            return c

        lax.fori_loop(0, NB, block_body, jnp.int32(0))

        tvs, tis = out_slices(NB - 2)
        pltpu.make_async_copy(otv0.at[pl.ds(0, RB * K)], tvs, osem0).wait()
        pltpu.make_async_copy(oti0.at[pl.ds(0, RB * K)], tis, osem0).wait()
        tvs, tis = out_slices(NB - 1)
        pltpu.make_async_copy(otv1.at[pl.ds(0, RB * K)], tvs, osem1).wait()
        pltpu.make_async_copy(oti1.at[pl.ds(0, RB * K)], tis, osem1).wait()
        gather_wait(gix0, grv0, gsem0)
        pltpu.sync_copy(grv0, rv_slice(NB - 2))
        gather_wait(gix1, grv1, gsem1)
        pltpu.sync_copy(grv1, rv_slice(NB - 1))

    return tk(simt2, mv2)


def _attn_mem_kernel(q_ref, k_ref, v_ref, tv_ref, rv_ref, o_ref):
    i = pl.program_id(1)
    outs = []
    for j in range(HP):
        sl = slice(j * DH, (j + 1) * DH)
        s = _causal_scores(q_ref[:, sl], k_ref[:, sl], i)
        tv = tv_ref[j]
        m = jnp.maximum(jnp.max(s, axis=-1, keepdims=True),
                        jnp.max(tv, axis=-1, keepdims=True))
        p = jnp.exp(s - m)
        w = jnp.exp(tv - m)
        l = (jnp.sum(p, axis=-1, keepdims=True)
             + jnp.sum(w, axis=-1, keepdims=True))
        acc = jnp.dot(p, v_ref[:, sl], preferred_element_type=jnp.float32)
        for kk in range(K):
            acc = acc + w[:, kk:kk + 1] * rv_ref[j, :, kk, :]
        outs.append(acc / l)
    o_ref[...] = jnp.concatenate(outs, axis=-1)


def _attn_with_mem(q, k, v, mem_scores, retrieved):
    head_row = pl.BlockSpec((QB, HP * DH), lambda h, i: (i, h))
    head_full = pl.BlockSpec((S, HP * DH), lambda h, i: (0, h))
    return pl.pallas_call(
        _attn_mem_kernel,
        grid=(H // HP, NQ),
        in_specs=[head_row, head_full, head_full,
                  pl.BlockSpec((HP, QB, K), lambda h, i: (h, i, 0)),
                  pl.BlockSpec((HP, QB, K, DH), lambda h, i: (h, i, 0, 0))],
        out_specs=head_row,
        out_shape=jax.ShapeDtypeStruct((S, D), jnp.float32),
    )(q, k, v, mem_scores, retrieved)


def _proj_ffn_kernel(a_ref, x_ref, wo_ref, g2_ref, w1_ref, w2_ref, o_ref):
    xx = x_ref[...] + jnp.dot(a_ref[...], wo_ref[...],
                              preferred_element_type=jnp.float32)
    mu = jnp.mean(xx, axis=-1, keepdims=True)
    var = jnp.mean((xx - mu) ** 2, axis=-1, keepdims=True)
    h2 = (xx - mu) * jax.lax.rsqrt(var + 1e-5) * g2_ref[...]
    t = jax.nn.gelu(jnp.dot(h2, w1_ref[...], preferred_element_type=jnp.float32))
    o_ref[...] = xx + jnp.dot(t, w2_ref[...], preferred_element_type=jnp.float32)


def _proj_ffn(attn_out, x, wo, g2, w1, w2):
    row = pl.BlockSpec((QB, D), lambda i: (i, 0))
    return pl.pallas_call(
        _proj_ffn_kernel,
        grid=(NQ,),
        in_specs=[row, row,
                  pl.BlockSpec((D, D), lambda i: (0, 0)),
                  pl.BlockSpec((1, D), lambda i: (0, 0)),
                  pl.BlockSpec((D, FF), lambda i: (0, 0)),
                  pl.BlockSpec((FF, D), lambda i: (0, 0))],
        out_specs=row,
        out_shape=jax.ShapeDtypeStruct((S, D), jnp.float32),
    )(attn_out, x, wo, g2, w1, w2)


def kernel(x, batch_indices, mem_k, mem_v, ln1, wq, wk, wv, wo, gate, ln2, w1, w2):
    xx = x[0]
    mk2 = mem_k[0].reshape(M, D)

    for l in range(L):
        q, k, v = _qkv(xx, ln1[l][None], wq[l], wk[l], wv[l])
        if l == 1:
            simt = _sim_mem(q, mk2)
            tv_f, ti_f, rv_f = _topk_sc(simt.reshape(NR * MW),
                                        mem_v[0].reshape(M * H, DH))
            top_vals = tv_f.reshape(H, S, K)
            retrieved = rv_f.reshape(H, S, K, DH)
            mem_scores = top_vals + gate[l][:, None, None]
            attn_out = _attn_with_mem(q, k, v, mem_scores, retrieved)
        else:
            attn_out = _attn_local(q, k, v)
        xx = _proj_ffn(attn_out, xx, wo[l], ln2[l][None], w1[l], w2[l])
    return xx[None]
